# baseline (speedup 1.0000x reference)
import functools

import jax
import jax.numpy as jnp
from jax import lax
from jax.experimental import pallas as pl
from jax.experimental.pallas import tpu as pltpu
from jax.experimental.pallas import tpu_sc as plsc

B = 16384
F_CAT = 26
F_REAL = 13
V = 100000
D = 32
F_TOT = F_CAT + F_REAL

NC = 2
NS = 16
NW = NC * NS
BPW = B // NW
GCH = 128
NCH = BPW // GCH


def _sc_gather_body(uc_hbm, tab_hbm, out_hbm, idx2_v, sup_v, sub_v, gbuf_v,
                    cbuf_v, gsem, osem):
    c = lax.axis_index("c")
    s = lax.axis_index("s")
    wid = s * NC + c
    base = wid * BPW
    pltpu.sync_copy(uc_hbm.at[pl.ds(base, BPW)], idx2_v)

    lanes = lax.broadcasted_iota(jnp.int32, (16,), 0)
    zeros = jnp.zeros((16,), jnp.int32)

    def ext_body(t, _):
        f = t // (BPW // 16)
        j = t % (BPW // 16)
        rows = lanes + j * 16
        fcol = zeros + f
        v = plsc.load_gather(idx2_v, [rows, fcol])
        sup_v[f, pl.ds(j * 16, 16)] = (v >> 2) + f * (V // 4)
        sub_v[f, pl.ds(j * 16, 16)] = v & 3
        return 0

    lax.fori_loop(0, F_CAT * (BPW // 16), ext_body, 0)

    NK = F_CAT * NCH

    def issue_gather(k):
        f = k // NCH
        ch = k - f * NCH
        pltpu.async_copy(
            tab_hbm.at[sup_v.at[f, pl.ds(ch * GCH, GCH)]],
            gbuf_v.at[k & 1],
            gsem,
        )

    def wait_gather(k):
        pltpu.make_async_copy(
            tab_hbm.at[sup_v.at[0, pl.ds(0, GCH)]], gbuf_v.at[k & 1], gsem
        ).wait()

    def issue_out(f):
        pltpu.async_copy(
            cbuf_v.at[f & 1], out_hbm.at[pl.ds(base, BPW), f], osem
        )

    def wait_out(f):
        pltpu.make_async_copy(
            cbuf_v.at[f & 1], out_hbm.at[pl.ds(base, BPW), f], osem
        ).wait()

    def compact(k):
        f = k // NCH
        ch = k - f * NCH
        gpar = k & 1
        fpar = f & 1

        def row_body(i, _):
            sv = sub_v[f, pl.ds(ch * GCH + i * 16, 16)]
            for u in range(16):
                r = i * 16 + u
                off = sv[u] * D
                cbuf_v[fpar, ch * GCH + r, pl.ds(0, 16)] = (
                    gbuf_v[gpar, r, pl.ds(off, 16)])
                cbuf_v[fpar, ch * GCH + r, pl.ds(16, 16)] = (
                    gbuf_v[gpar, r, pl.ds(off + 16, 16)])
            return 0

        lax.fori_loop(0, GCH // 16, row_body, 0)

    def pipe_body(k, _):
        issue_gather(k)

        @pl.when(k >= 1)
        def _():
            km = k - 1
            f = km // NCH
            ch = km - f * NCH
            wait_gather(km)

            @pl.when(jnp.logical_and(ch == 0, f >= 2))
            def _():
                wait_out(f - 2)

            compact(km)

            @pl.when(ch == NCH - 1)
            def _():
                issue_out(f)

        return 0

    lax.fori_loop(0, NK, pipe_body, 0)
    wait_gather(NK - 1)
    compact(NK - 1)
    issue_out(F_CAT - 1)
    wait_out(F_CAT - 2)
    wait_out(F_CAT - 1)


_sc_gather = functools.partial(
    pl.kernel,
    out_type=jax.ShapeDtypeStruct((B, F_TOT, D), jnp.float32),
    mesh=plsc.VectorSubcoreMesh(core_axis_name="c", subcore_axis_name="s"),
    scratch_types=[
        pltpu.VMEM((BPW, F_CAT), jnp.int32),
        pltpu.VMEM((F_CAT, BPW), jnp.int32),
        pltpu.VMEM((F_CAT, BPW), jnp.int32),
        pltpu.VMEM((2, GCH, 4 * D), jnp.float32),
        pltpu.VMEM((2, BPW, D), jnp.float32),
        pltpu.SemaphoreType.DMA,
        pltpu.SemaphoreType.DMA,
    ],
    compiler_params=pltpu.CompilerParams(
        use_tc_tiling_on_sc=False, needs_layout_passes=False
    ),
)(_sc_gather_body)


def _tc_sweep_body(x_ref, w_ref, b_ref, g_ref, be_ref, t_ref, io_ref, out_ref):
    cat = io_ref[:, :F_CAT, :] + t_ref[...][None, :F_CAT, :]
    x = x_ref[...]
    w = w_ref[...]
    b = b_ref[...]
    h = x[:, :, None] * w[None] + b[None]
    mu = jnp.mean(h, axis=-1, keepdims=True)
    var = jnp.mean((h - mu) * (h - mu), axis=-1, keepdims=True)
    h = (h - mu) * lax.rsqrt(var + 1e-5)
    h = h * g_ref[...][None] + be_ref[...][None]
    h = jnp.maximum(h, 0.0)
    real = h + t_ref[...][None, F_CAT:, :]
    out_ref[...] = jnp.concatenate([cat, real], axis=1)


BBLK = 512


def _tc_sweep(ur, real_w, real_b, ln_gamma, ln_beta, type_emb, combined):
    return pl.pallas_call(
        _tc_sweep_body,
        out_shape=jax.ShapeDtypeStruct((B, F_TOT, D), jnp.float32),
        grid=(B // BBLK,),
        in_specs=[
            pl.BlockSpec((BBLK, F_REAL), lambda i: (i, 0)),
            pl.BlockSpec((F_REAL, D), lambda i: (0, 0)),
            pl.BlockSpec((F_REAL, D), lambda i: (0, 0)),
            pl.BlockSpec((F_REAL, D), lambda i: (0, 0)),
            pl.BlockSpec((F_REAL, D), lambda i: (0, 0)),
            pl.BlockSpec((F_TOT, D), lambda i: (0, 0)),
            pl.BlockSpec((BBLK, F_TOT, D), lambda i: (i, 0, 0)),
        ],
        out_specs=pl.BlockSpec((BBLK, F_TOT, D), lambda i: (i, 0, 0)),
        input_output_aliases={6: 0},
    )(ur, real_w, real_b, ln_gamma, ln_beta, type_emb, combined)


def kernel(user_categoricals, user_reals, cat_tables, type_emb, real_w, real_b,
           ln_gamma, ln_beta):
    tab4 = cat_tables.reshape(F_CAT * V // 4, 4 * D)
    combined = _sc_gather(user_categoricals, tab4)
    return _tc_sweep(user_reals, real_w, real_b, ln_gamma, ln_beta,
                     type_emb, combined)


# speedup vs baseline: 1.2370x; 1.2370x over previous
import functools

import jax
import jax.numpy as jnp
from jax import lax
from jax.experimental import pallas as pl
from jax.experimental.pallas import tpu as pltpu
from jax.experimental.pallas import tpu_sc as plsc

B = 16384
F_CAT = 26
F_REAL = 13
V = 100000
D = 32
F_TOT = F_CAT + F_REAL
FP = 40
DP = 128

NC = 2
NS = 16
NW = NC * NS
BPW = B // NW
GCH = 128
NCH = BPW // GCH
NK = F_CAT * NCH


def _sc_gather_body(uc_hbm, tab_hbm, out_hbm, idx2_v, idxs_v, gbuf_v, gsem, osem):
    c = lax.axis_index("c")
    s = lax.axis_index("s")
    wid = s * NC + c
    base = wid * BPW
    pltpu.sync_copy(uc_hbm.at[pl.ds(base, BPW)], idx2_v)

    lanes = lax.broadcasted_iota(jnp.int32, (16,), 0)
    zeros = jnp.zeros((16,), jnp.int32)

    def ext_body(t, _):
        f = t // (BPW // 16)
        j = t % (BPW // 16)
        rows = lanes + j * 16
        fcol = zeros + f
        v = plsc.load_gather(idx2_v, [rows, fcol])
        idxs_v[f, pl.ds(j * 16, 16)] = v + f * V
        return 0

    lax.fori_loop(0, F_CAT * (BPW // 16), ext_body, 0)

    def issue_gather(k):
        f = k // NCH
        ch = k - f * NCH
        pltpu.async_copy(
            tab_hbm.at[idxs_v.at[f, pl.ds(ch * GCH, GCH)]],
            gbuf_v.at[k & 1],
            gsem,
        )

    def wait_gather(k):
        pltpu.make_async_copy(
            tab_hbm.at[idxs_v.at[0, pl.ds(0, GCH)]], gbuf_v.at[k & 1], gsem
        ).wait()

    def out_slice(k):
        f = k // NCH
        ch = k - f * NCH
        return out_hbm.at[pl.ds(base + ch * GCH, GCH), f, pl.ds(0, D)]

    def issue_out(k):
        pltpu.async_copy(gbuf_v.at[k & 1], out_slice(k), osem)

    def wait_out(k):
        pltpu.make_async_copy(gbuf_v.at[k & 1], out_slice(k), osem).wait()

    def pipe_body(k, _):
        @pl.when(k >= 2)
        def _():
            wait_out(k - 2)

        issue_gather(k)

        @pl.when(k >= 1)
        def _():
            wait_gather(k - 1)
            issue_out(k - 1)

        return 0

    lax.fori_loop(0, NK, pipe_body, 0)
    wait_gather(NK - 1)
    issue_out(NK - 1)
    wait_out(NK - 2)
    wait_out(NK - 1)


_sc_gather = functools.partial(
    pl.kernel,
    out_type=jax.ShapeDtypeStruct((B, FP, DP), jnp.float32),
    mesh=plsc.VectorSubcoreMesh(core_axis_name="c", subcore_axis_name="s"),
    scratch_types=[
        pltpu.VMEM((BPW, F_CAT), jnp.int32),
        pltpu.VMEM((F_CAT, BPW), jnp.int32),
        pltpu.VMEM((2, GCH, D), jnp.float32),
        pltpu.SemaphoreType.DMA,
        pltpu.SemaphoreType.DMA,
    ],
    compiler_params=pltpu.CompilerParams(
        use_tc_tiling_on_sc=False, needs_layout_passes=False
    ),
)(_sc_gather_body)


def _tc_sweep_body(x_ref, w_ref, b_ref, g_ref, be_ref, t_ref, io_ref, out_ref):
    io = io_ref[...]
    t128 = t_ref[...]
    cat = io[:, :F_CAT, :] + t128[None, :F_CAT, :]
    x = x_ref[...]
    w = w_ref[...]
    b = b_ref[...]
    h = x[:, :, None] * w[None] + b[None]
    mu = jnp.mean(h, axis=-1, keepdims=True)
    var = jnp.mean((h - mu) * (h - mu), axis=-1, keepdims=True)
    h = (h - mu) * lax.rsqrt(var + 1e-5)
    h = h * g_ref[...][None] + be_ref[...][None]
    h = jnp.maximum(h, 0.0)
    real = h + t128[None, F_CAT:F_TOT, :D]
    real128 = jnp.concatenate(
        [real, jnp.zeros((real.shape[0], F_REAL, DP - D), jnp.float32)], axis=-1)
    out_ref[...] = jnp.concatenate(
        [cat, real128, io[:, F_TOT:, :]], axis=1)


BBLK = 256


def _tc_sweep(ur, real_w, real_b, ln_gamma, ln_beta, t128, combined):
    return pl.pallas_call(
        _tc_sweep_body,
        out_shape=jax.ShapeDtypeStruct((B, FP, DP), jnp.float32),
        grid=(B // BBLK,),
        in_specs=[
            pl.BlockSpec((BBLK, F_REAL), lambda i: (i, 0)),
            pl.BlockSpec((F_REAL, D), lambda i: (0, 0)),
            pl.BlockSpec((F_REAL, D), lambda i: (0, 0)),
            pl.BlockSpec((F_REAL, D), lambda i: (0, 0)),
            pl.BlockSpec((F_REAL, D), lambda i: (0, 0)),
            pl.BlockSpec((FP, DP), lambda i: (0, 0)),
            pl.BlockSpec((BBLK, FP, DP), lambda i: (i, 0, 0)),
        ],
        out_specs=pl.BlockSpec((BBLK, FP, DP), lambda i: (i, 0, 0)),
        input_output_aliases={6: 0},
    )(ur, real_w, real_b, ln_gamma, ln_beta, t128, combined)


def kernel(user_categoricals, user_reals, cat_tables, type_emb, real_w, real_b,
           ln_gamma, ln_beta):
    tab2 = cat_tables.reshape(F_CAT * V, D)
    combined = _sc_gather(user_categoricals, tab2)
    t128 = jnp.pad(type_emb, ((0, FP - F_TOT), (0, DP - D)))
    swept = _tc_sweep(user_reals, real_w, real_b, ln_gamma, ln_beta,
                      t128, combined)
    return swept[:, :F_TOT, :D]
